# Initial kernel scaffold; baseline (speedup 1.0000x reference)
import jax
import jax.numpy as jnp
from jax.experimental import pallas as pl


def kernel(x, edge_index, W1, b1, W2, b2):
    raise NotImplementedError("write your pallas kernel here")


# speedup vs baseline: 5.5553x; 5.5553x over previous
import functools

import jax
import jax.numpy as jnp
from jax import lax
from jax.experimental import pallas as pl
from jax.experimental.pallas import tpu as pltpu
from jax.experimental.pallas import tpu_sc as plsc

N = 10000
E = 320000
D = 128

NC = 2
NS = 16
NW = NC * NS
EPT = E // NW
C = 80
NCH = EPT // C
G = 25
NG = NCH // G
RPT = 640
RPT_LAST = N - (NS - 1) * RPT
BN = 1280
GRID = 8


def _mesh():
    return plsc.VectorSubcoreMesh(core_axis_name="c", subcore_axis_name="s",
                                  num_cores=NC, num_subcores=NS)


def _tile_ids():
    c = lax.axis_index("c")
    s = lax.axis_index("s")
    return c, s, s * NC + c


def _rowwise(fn):
    def run(s, r0):
        pl.when(s < NS - 1)(lambda: fn(r0, RPT))
        pl.when(s == NS - 1)(lambda: fn(r0, RPT_LAST))
    return run


@functools.cache
def _make_segsum():
    scratch = [
        pltpu.VMEM((G, C), jnp.int32),
        pltpu.VMEM((G, C), jnp.int32),
        pltpu.VMEM((C, D), jnp.float32),
        pltpu.VMEM_SHARED((N, D), jnp.float32),
        pltpu.SemaphoreType.DMA,
    ]

    def body(y_hbm, src_hbm, dst_hbm, z_hbm, out_hbm,
             src_v, dst_v, rows_v, acc_sh, sem):
        c, s, wid = _tile_ids()
        r0 = pl.multiple_of(s * RPT, 8)

        def init_rows(rr, nr):
            pltpu.sync_copy(z_hbm.at[pl.ds(rr, nr)], acc_sh.at[pl.ds(rr, nr)])

        _rowwise(init_rows)(s, r0)
        plsc.subcore_barrier()

        def group(g, carry):
            pltpu.sync_copy(src_hbm.at[wid, g], src_v)
            pltpu.sync_copy(dst_hbm.at[wid, g], dst_v)

            def chunk(j, carry2):
                pltpu.async_copy(y_hbm.at[src_v.at[j]], rows_v, sem).wait()
                pltpu.sync_copy(rows_v, acc_sh.at[dst_v.at[j]], add=True)
                return carry2

            return lax.fori_loop(0, G, chunk, carry)

        lax.fori_loop(0, NG, group, 0)
        plsc.subcore_barrier()

        def write_rows(rr, nr):
            pltpu.sync_copy(acc_sh.at[pl.ds(rr, nr)],
                            out_hbm.at[c, pl.ds(rr, nr), :])

        _rowwise(write_rows)(s, r0)

    return pl.kernel(
        body,
        out_type=jax.ShapeDtypeStruct((NC, N, D), jnp.float32),
        mesh=_mesh(),
        scratch_types=scratch,
    )


def _segsum(*args):
    return _make_segsum()(*args)


@functools.cache
def _make_deg():
    scratch = [
        pltpu.VMEM((G, C), jnp.int32),
        pltpu.VMEM((C, D), jnp.float32),
        pltpu.VMEM_SHARED((N, D), jnp.float32),
    ]

    def body(dst_hbm, z_hbm, on_hbm, out_hbm, dst_v, ones_v, acc_sh):
        c, s, wid = _tile_ids()
        r0 = pl.multiple_of(s * RPT, 8)

        def init_rows(rr, nr):
            pltpu.sync_copy(z_hbm.at[pl.ds(rr, nr)], acc_sh.at[pl.ds(rr, nr)])

        _rowwise(init_rows)(s, r0)
        pltpu.sync_copy(on_hbm, ones_v)
        plsc.subcore_barrier()

        def group(g, carry):
            pltpu.sync_copy(dst_hbm.at[wid, g], dst_v)

            def chunk(j, carry2):
                pltpu.sync_copy(ones_v, acc_sh.at[dst_v.at[j]], add=True)
                return carry2

            return lax.fori_loop(0, G, chunk, carry)

        lax.fori_loop(0, NG, group, 0)
        plsc.subcore_barrier()

        def write_rows(rr, nr):
            pltpu.sync_copy(acc_sh.at[pl.ds(rr, nr)],
                            out_hbm.at[c, pl.ds(rr, nr), :])

        _rowwise(write_rows)(s, r0)

    return pl.kernel(
        body,
        out_type=jax.ShapeDtypeStruct((NC, N, D), jnp.float32),
        mesh=_mesh(),
        scratch_types=scratch,
    )


def _deg(*args):
    return _make_deg()(*args)


def _mm_body(x_ref, w_ref, o_ref):
    o_ref[...] = jnp.dot(x_ref[...], w_ref[...],
                         preferred_element_type=jnp.float32,
                         precision=lax.Precision.HIGHEST)


def _tc_matmul(x, W):
    return pl.pallas_call(
        _mm_body,
        grid=(GRID,),
        in_specs=[pl.BlockSpec((BN, D), lambda i: (i, 0)),
                  pl.BlockSpec((D, D), lambda i: (0, 0))],
        out_specs=pl.BlockSpec((BN, D), lambda i: (i, 0)),
        out_shape=jax.ShapeDtypeStruct((N, D), jnp.float32),
    )(x, W)


def _ep1_body(a_ref, g_ref, y_ref, b_ref, w_ref, o_ref):
    agg = a_ref[0] + a_ref[1] + y_ref[...]
    den = g_ref[0, :, 0:1] + g_ref[1, :, 0:1] + 1.0
    h = agg / den + b_ref[...]
    h = jnp.maximum(h, 0.0)
    o_ref[...] = jnp.dot(h, w_ref[...],
                         preferred_element_type=jnp.float32,
                         precision=lax.Precision.HIGHEST)


def _tc_ep1(a, dp, y, b1, W2):
    return pl.pallas_call(
        _ep1_body,
        grid=(GRID,),
        in_specs=[pl.BlockSpec((2, BN, D), lambda i: (0, i, 0)),
                  pl.BlockSpec((2, BN, D), lambda i: (0, i, 0)),
                  pl.BlockSpec((BN, D), lambda i: (i, 0)),
                  pl.BlockSpec((1, D), lambda i: (0, 0)),
                  pl.BlockSpec((D, D), lambda i: (0, 0))],
        out_specs=pl.BlockSpec((BN, D), lambda i: (i, 0)),
        out_shape=jax.ShapeDtypeStruct((N, D), jnp.float32),
    )(a, dp, y, b1.reshape(1, D), W2)


def _ep2_body(a_ref, g_ref, y_ref, b_ref, o_ref):
    agg = a_ref[0] + a_ref[1] + y_ref[...]
    den = g_ref[0, :, 0:1] + g_ref[1, :, 0:1] + 1.0
    o_ref[...] = agg / den + b_ref[...]


def _tc_ep2(a, dp, y, b2):
    return pl.pallas_call(
        _ep2_body,
        grid=(GRID,),
        in_specs=[pl.BlockSpec((2, BN, D), lambda i: (0, i, 0)),
                  pl.BlockSpec((2, BN, D), lambda i: (0, i, 0)),
                  pl.BlockSpec((BN, D), lambda i: (i, 0)),
                  pl.BlockSpec((1, D), lambda i: (0, 0))],
        out_specs=pl.BlockSpec((BN, D), lambda i: (i, 0)),
        out_shape=jax.ShapeDtypeStruct((N, D), jnp.float32),
    )(a, dp, y, b2.reshape(1, D))


def kernel(x, edge_index, W1, b1, W2, b2):
    src = edge_index[0].reshape(NW, NG, G, C)
    dst = edge_index[1].reshape(NW, NG, G, C)
    z = jnp.zeros((N, D), jnp.float32)
    on = jnp.ones((C, D), jnp.float32)

    dp = _deg(dst, z, on)
    y1 = _tc_matmul(x, W1)
    a1 = _segsum(y1, src, dst, z)
    y2 = _tc_ep1(a1, dp, y1, b1, W2)
    a2 = _segsum(y2, src, dst, z)
    return _tc_ep2(a2, dp, y2, b2)


# speedup vs baseline: 7.9315x; 1.4277x over previous
import functools

import jax
import jax.numpy as jnp
from jax import lax
from jax.experimental import pallas as pl
from jax.experimental.pallas import tpu as pltpu
from jax.experimental.pallas import tpu_sc as plsc

N = 10000
E = 320000
D = 128

NC = 2
NS = 16
NW = NC * NS
EPT = E // NW
C = 80
NCH = EPT // C
G = 25
NG = NCH // G
RPT = 640
RPT_LAST = N - (NS - 1) * RPT
BN = 1280
GRID = 8


def _mesh():
    return plsc.VectorSubcoreMesh(core_axis_name="c", subcore_axis_name="s",
                                  num_cores=NC, num_subcores=NS)


def _tile_ids():
    c = lax.axis_index("c")
    s = lax.axis_index("s")
    return c, s, s * NC + c


def _rowwise(fn):
    def run(s, r0):
        pl.when(s < NS - 1)(lambda: fn(r0, RPT))
        pl.when(s == NS - 1)(lambda: fn(r0, RPT_LAST))
    return run


@functools.cache
def _make_segsum():
    scratch = [
        pltpu.VMEM((G, C), jnp.int32),
        pltpu.VMEM((G, C), jnp.int32),
        pltpu.VMEM((2, C, D), jnp.float32),
        pltpu.VMEM_SHARED((N, D), jnp.float32),
        pltpu.SemaphoreType.DMA,
    ]

    def body(y_hbm, src_hbm, dst_hbm, z_hbm, out_hbm,
             src_v, dst_v, rows_v, acc_sh, sem):
        c, s, wid = _tile_ids()
        r0 = pl.multiple_of(s * RPT, 8)

        def init_rows(rr, nr):
            pltpu.sync_copy(z_hbm.at[pl.ds(rr, nr)], acc_sh.at[pl.ds(rr, nr)])

        _rowwise(init_rows)(s, r0)
        plsc.subcore_barrier()

        def group(g, carry):
            pltpu.sync_copy(src_hbm.at[wid, g], src_v)
            pltpu.sync_copy(dst_hbm.at[wid, g], dst_v)

            pltpu.async_copy(y_hbm.at[src_v.at[0]], rows_v.at[0], sem)

            def chunk(j, carry2):
                p = lax.rem(j, 2)
                q = lax.rem(j + 1, 2)

                @pl.when(j < G - 1)
                def _():
                    pltpu.async_copy(y_hbm.at[src_v.at[j + 1]],
                                     rows_v.at[q], sem)

                pltpu.make_async_copy(y_hbm.at[src_v.at[j]],
                                      rows_v.at[p], sem).wait()
                pltpu.sync_copy(rows_v.at[p], acc_sh.at[dst_v.at[j]], add=True)
                return carry2

            return lax.fori_loop(0, G, chunk, carry)

        lax.fori_loop(0, NG, group, 0)
        plsc.subcore_barrier()

        def write_rows(rr, nr):
            pltpu.sync_copy(acc_sh.at[pl.ds(rr, nr)],
                            out_hbm.at[c, pl.ds(rr, nr), :])

        _rowwise(write_rows)(s, r0)

    return pl.kernel(
        body,
        out_type=jax.ShapeDtypeStruct((NC, N, D), jnp.float32),
        mesh=_mesh(),
        scratch_types=scratch,
    )


def _segsum(*args):
    return _make_segsum()(*args)


@functools.cache
def _make_deg():
    scratch = [
        pltpu.VMEM((G, C), jnp.int32),
        pltpu.VMEM((C, D), jnp.float32),
        pltpu.VMEM_SHARED((N, D), jnp.float32),
    ]

    def body(dst_hbm, z_hbm, on_hbm, out_hbm, dst_v, ones_v, acc_sh):
        c, s, wid = _tile_ids()
        r0 = pl.multiple_of(s * RPT, 8)

        def init_rows(rr, nr):
            pltpu.sync_copy(z_hbm.at[pl.ds(rr, nr)], acc_sh.at[pl.ds(rr, nr)])

        _rowwise(init_rows)(s, r0)
        pltpu.sync_copy(on_hbm, ones_v)
        plsc.subcore_barrier()

        def group(g, carry):
            pltpu.sync_copy(dst_hbm.at[wid, g], dst_v)

            def chunk(j, carry2):
                pltpu.sync_copy(ones_v, acc_sh.at[dst_v.at[j]], add=True)
                return carry2

            return lax.fori_loop(0, G, chunk, carry)

        lax.fori_loop(0, NG, group, 0)
        plsc.subcore_barrier()

        def write_rows(rr, nr):
            pltpu.sync_copy(acc_sh.at[pl.ds(rr, nr)],
                            out_hbm.at[c, pl.ds(rr, nr), :])

        _rowwise(write_rows)(s, r0)

    return pl.kernel(
        body,
        out_type=jax.ShapeDtypeStruct((NC, N, D), jnp.float32),
        mesh=_mesh(),
        scratch_types=scratch,
    )


def _deg(*args):
    return _make_deg()(*args)


def _mm_body(x_ref, w_ref, o_ref):
    o_ref[...] = jnp.dot(x_ref[...], w_ref[...],
                         preferred_element_type=jnp.float32,
                         precision=lax.Precision.HIGHEST)


def _tc_matmul(x, W):
    return pl.pallas_call(
        _mm_body,
        grid=(GRID,),
        in_specs=[pl.BlockSpec((BN, D), lambda i: (i, 0)),
                  pl.BlockSpec((D, D), lambda i: (0, 0))],
        out_specs=pl.BlockSpec((BN, D), lambda i: (i, 0)),
        out_shape=jax.ShapeDtypeStruct((N, D), jnp.float32),
    )(x, W)


def _ep1_body(a_ref, g_ref, y_ref, b_ref, w_ref, o_ref):
    agg = a_ref[0] + a_ref[1] + y_ref[...]
    den = g_ref[0, :, 0:1] + g_ref[1, :, 0:1] + 1.0
    h = agg / den + b_ref[...]
    h = jnp.maximum(h, 0.0)
    o_ref[...] = jnp.dot(h, w_ref[...],
                         preferred_element_type=jnp.float32,
                         precision=lax.Precision.HIGHEST)


def _tc_ep1(a, dp, y, b1, W2):
    return pl.pallas_call(
        _ep1_body,
        grid=(GRID,),
        in_specs=[pl.BlockSpec((2, BN, D), lambda i: (0, i, 0)),
                  pl.BlockSpec((2, BN, D), lambda i: (0, i, 0)),
                  pl.BlockSpec((BN, D), lambda i: (i, 0)),
                  pl.BlockSpec((1, D), lambda i: (0, 0)),
                  pl.BlockSpec((D, D), lambda i: (0, 0))],
        out_specs=pl.BlockSpec((BN, D), lambda i: (i, 0)),
        out_shape=jax.ShapeDtypeStruct((N, D), jnp.float32),
    )(a, dp, y, b1.reshape(1, D), W2)


def _ep2_body(a_ref, g_ref, y_ref, b_ref, o_ref):
    agg = a_ref[0] + a_ref[1] + y_ref[...]
    den = g_ref[0, :, 0:1] + g_ref[1, :, 0:1] + 1.0
    o_ref[...] = agg / den + b_ref[...]


def _tc_ep2(a, dp, y, b2):
    return pl.pallas_call(
        _ep2_body,
        grid=(GRID,),
        in_specs=[pl.BlockSpec((2, BN, D), lambda i: (0, i, 0)),
                  pl.BlockSpec((2, BN, D), lambda i: (0, i, 0)),
                  pl.BlockSpec((BN, D), lambda i: (i, 0)),
                  pl.BlockSpec((1, D), lambda i: (0, 0))],
        out_specs=pl.BlockSpec((BN, D), lambda i: (i, 0)),
        out_shape=jax.ShapeDtypeStruct((N, D), jnp.float32),
    )(a, dp, y, b2.reshape(1, D))


def kernel(x, edge_index, W1, b1, W2, b2):
    src = edge_index[0].reshape(NW, NG, G, C)
    dst = edge_index[1].reshape(NW, NG, G, C)
    z = jnp.zeros((N, D), jnp.float32)
    on = jnp.ones((C, D), jnp.float32)

    dp = _deg(dst, z, on)
    y1 = _tc_matmul(x, W1)
    a1 = _segsum(y1, src, dst, z)
    y2 = _tc_ep1(a1, dp, y1, b1, W2)
    a2 = _segsum(y2, src, dst, z)
    return _tc_ep2(a2, dp, y2, b2)


# speedup vs baseline: 8.8948x; 1.1214x over previous
import functools

import jax
import jax.numpy as jnp
from jax import lax
from jax.experimental import pallas as pl
from jax.experimental.pallas import tpu as pltpu
from jax.experimental.pallas import tpu_sc as plsc

N = 10000
E = 320000
D = 128

NC = 2
NS = 16
NW = NC * NS
EPT = E // NW
C = 80
NCH = EPT // C
G = 25
NG = NCH // G
RPT = 640
RPT_LAST = N - (NS - 1) * RPT
BN = 1280
GRID = 8


def _mesh():
    return plsc.VectorSubcoreMesh(core_axis_name="c", subcore_axis_name="s",
                                  num_cores=NC, num_subcores=NS)


def _tile_ids():
    c = lax.axis_index("c")
    s = lax.axis_index("s")
    return c, s, s * NC + c


def _rowwise(fn):
    def run(s, r0):
        pl.when(s < NS - 1)(lambda: fn(r0, RPT))
        pl.when(s == NS - 1)(lambda: fn(r0, RPT_LAST))
    return run


@functools.cache
def _make_segsum():
    scratch = [
        pltpu.VMEM((G, C), jnp.int32),
        pltpu.VMEM((G, C), jnp.int32),
        pltpu.VMEM((3, C, D), jnp.float32),
        pltpu.VMEM_SHARED((N, D), jnp.float32),
        pltpu.SemaphoreType.DMA,
    ]

    def body(y_hbm, src_hbm, dst_hbm, z_hbm, out_hbm,
             src_v, dst_v, rows_v, acc_sh, sem):
        c, s, wid = _tile_ids()
        r0 = pl.multiple_of(s * RPT, 8)

        def init_rows(rr, nr):
            pltpu.sync_copy(z_hbm.at[pl.ds(rr, nr)], acc_sh.at[pl.ds(rr, nr)])

        _rowwise(init_rows)(s, r0)
        plsc.subcore_barrier()

        def group(g, carry):
            pltpu.sync_copy(src_hbm.at[wid, g], src_v)
            pltpu.sync_copy(dst_hbm.at[wid, g], dst_v)

            pltpu.async_copy(y_hbm.at[src_v.at[0]], rows_v.at[0], sem)
            pltpu.async_copy(y_hbm.at[src_v.at[1]], rows_v.at[1], sem)

            def chunk(j, carry2):
                p = lax.rem(j, 3)
                q = lax.rem(j + 2, 3)

                @pl.when(j < G - 2)
                def _():
                    pltpu.async_copy(y_hbm.at[src_v.at[j + 2]],
                                     rows_v.at[q], sem)

                pltpu.make_async_copy(y_hbm.at[src_v.at[j]],
                                      rows_v.at[p], sem).wait()
                pltpu.sync_copy(rows_v.at[p], acc_sh.at[dst_v.at[j]], add=True)
                return carry2

            return lax.fori_loop(0, G, chunk, carry)

        lax.fori_loop(0, NG, group, 0)
        plsc.subcore_barrier()

        def write_rows(rr, nr):
            pltpu.sync_copy(acc_sh.at[pl.ds(rr, nr)],
                            out_hbm.at[c, pl.ds(rr, nr), :])

        _rowwise(write_rows)(s, r0)

    return pl.kernel(
        body,
        out_type=jax.ShapeDtypeStruct((NC, N, D), jnp.float32),
        mesh=_mesh(),
        scratch_types=scratch,
    )


def _segsum(*args):
    return _make_segsum()(*args)


DW = 128


@functools.cache
def _make_deg():
    scratch = [
        pltpu.VMEM((G, C), jnp.int32),
        pltpu.VMEM((C, DW), jnp.float32),
        pltpu.VMEM_SHARED((N, DW), jnp.float32),
    ]

    def body(dst_hbm, z_hbm, on_hbm, out_hbm, dst_v, ones_v, acc_sh):
        c, s, wid = _tile_ids()
        r0 = pl.multiple_of(s * RPT, 8)

        def init_rows(rr, nr):
            pltpu.sync_copy(z_hbm.at[pl.ds(rr, nr)], acc_sh.at[pl.ds(rr, nr)])

        _rowwise(init_rows)(s, r0)
        pltpu.sync_copy(on_hbm, ones_v)
        plsc.subcore_barrier()

        def group(g, carry):
            pltpu.sync_copy(dst_hbm.at[wid, g], dst_v)

            def chunk(j, carry2):
                pltpu.sync_copy(ones_v, acc_sh.at[dst_v.at[j]], add=True)
                return carry2

            return lax.fori_loop(0, G, chunk, carry)

        lax.fori_loop(0, NG, group, 0)
        plsc.subcore_barrier()

        def write_rows(rr, nr):
            pltpu.sync_copy(acc_sh.at[pl.ds(rr, nr)],
                            out_hbm.at[c, pl.ds(rr, nr), :])

        _rowwise(write_rows)(s, r0)

    return pl.kernel(
        body,
        out_type=jax.ShapeDtypeStruct((NC, N, DW), jnp.float32),
        mesh=_mesh(),
        scratch_types=scratch,
    )


def _deg(*args):
    return _make_deg()(*args)


def _mm_body(x_ref, w_ref, o_ref):
    o_ref[...] = jnp.dot(x_ref[...], w_ref[...],
                         preferred_element_type=jnp.float32,
                         precision=lax.Precision.HIGHEST)


def _tc_matmul(x, W):
    return pl.pallas_call(
        _mm_body,
        grid=(GRID,),
        in_specs=[pl.BlockSpec((BN, D), lambda i: (i, 0)),
                  pl.BlockSpec((D, D), lambda i: (0, 0))],
        out_specs=pl.BlockSpec((BN, D), lambda i: (i, 0)),
        out_shape=jax.ShapeDtypeStruct((N, D), jnp.float32),
    )(x, W)


def _ep1_body(a_ref, g_ref, y_ref, b_ref, w_ref, o_ref):
    agg = a_ref[0] + a_ref[1] + y_ref[...]
    den = g_ref[0, :, 0:1] + g_ref[1, :, 0:1] + 1.0
    h = agg / den + b_ref[...]
    h = jnp.maximum(h, 0.0)
    o_ref[...] = jnp.dot(h, w_ref[...],
                         preferred_element_type=jnp.float32,
                         precision=lax.Precision.HIGHEST)


def _tc_ep1(a, dp, y, b1, W2):
    return pl.pallas_call(
        _ep1_body,
        grid=(GRID,),
        in_specs=[pl.BlockSpec((2, BN, D), lambda i: (0, i, 0)),
                  pl.BlockSpec((2, BN, DW), lambda i: (0, i, 0)),
                  pl.BlockSpec((BN, D), lambda i: (i, 0)),
                  pl.BlockSpec((1, D), lambda i: (0, 0)),
                  pl.BlockSpec((D, D), lambda i: (0, 0))],
        out_specs=pl.BlockSpec((BN, D), lambda i: (i, 0)),
        out_shape=jax.ShapeDtypeStruct((N, D), jnp.float32),
    )(a, dp, y, b1.reshape(1, D), W2)


def _ep2_body(a_ref, g_ref, y_ref, b_ref, o_ref):
    agg = a_ref[0] + a_ref[1] + y_ref[...]
    den = g_ref[0, :, 0:1] + g_ref[1, :, 0:1] + 1.0
    o_ref[...] = agg / den + b_ref[...]


def _tc_ep2(a, dp, y, b2):
    return pl.pallas_call(
        _ep2_body,
        grid=(GRID,),
        in_specs=[pl.BlockSpec((2, BN, D), lambda i: (0, i, 0)),
                  pl.BlockSpec((2, BN, D), lambda i: (0, i, 0)),
                  pl.BlockSpec((BN, D), lambda i: (i, 0)),
                  pl.BlockSpec((1, D), lambda i: (0, 0))],
        out_specs=pl.BlockSpec((BN, D), lambda i: (i, 0)),
        out_shape=jax.ShapeDtypeStruct((N, D), jnp.float32),
    )(a, dp, y, b2.reshape(1, D))


def kernel(x, edge_index, W1, b1, W2, b2):
    src = edge_index[0].reshape(NW, NG, G, C)
    dst = edge_index[1].reshape(NW, NG, G, C)
    z = jnp.zeros((N, D), jnp.float32)
    on = jnp.ones((C, DW), jnp.float32)

    dp = _deg(dst, z, on)
    y1 = _tc_matmul(x, W1)
    a1 = _segsum(y1, src, dst, z)
    y2 = _tc_ep1(a1, dp, y1, b1, W2)
    a2 = _segsum(y2, src, dst, z)
    return _tc_ep2(a2, dp, y2, b2)
